# baseline (speedup 1.0000x reference)
import jax
import jax.numpy as jnp
from jax import lax
from jax.experimental import pallas as pl

_BLOCK = 2048


def _rvq_body(x_ref, cbt_ref, cbcat_ref, cb2_ref, idx_ref, q_ref):
    resid = x_ref[...]
    quant = jnp.zeros_like(resid)
    n_cb = cbt_ref.shape[0]
    b, k = resid.shape[0], cbt_ref.shape[2]
    cols = []
    for i in range(n_cb):
        a2 = jnp.sum(resid * resid, axis=1, keepdims=True)
        b2 = cb2_ref[i, :][None, :]
        t = lax.dot_general(resid, cbt_ref[i],
                            (((1,), (0,)), ((), ())),
                            preferred_element_type=jnp.float32)
        d2 = a2 + b2 + t
        dist = jnp.sqrt(jnp.maximum(d2, 0.0))
        m = jnp.min(dist, axis=1, keepdims=True)
        iota_k = lax.broadcasted_iota(jnp.int32, (b, k), 1)
        idx = jnp.min(jnp.where(dist == m, iota_k, k), axis=1)
        iota3 = lax.broadcasted_iota(jnp.int32, (b, 3 * k), 1)
        oh3 = ((iota3 & (k - 1)) == idx[:, None]).astype(jnp.bfloat16)
        q = lax.dot_general(oh3, cbcat_ref[i], (((1,), (0,)), ((), ())),
                            preferred_element_type=jnp.float32)
        quant = quant + q
        resid = resid - q
        cols.append(idx)
    idx_ref[...] = jnp.stack(cols, axis=1)
    q_ref[...] = quant


def kernel(x, codebooks):
    n, d = x.shape
    n_cb, k, _ = codebooks.shape
    cbt = -2.0 * jnp.swapaxes(codebooks, 1, 2)
    mask = jnp.uint32(0xFFFF0000)
    hi = lax.bitcast_convert_type(
        lax.bitcast_convert_type(codebooks, jnp.uint32) & mask, jnp.float32)
    r1 = codebooks - hi
    lo = lax.bitcast_convert_type(
        lax.bitcast_convert_type(r1, jnp.uint32) & mask, jnp.float32)
    lo2 = r1 - lo
    cbcat = jnp.concatenate(
        [hi.astype(jnp.bfloat16), lo.astype(jnp.bfloat16),
         lo2.astype(jnp.bfloat16)], axis=1)
    cb2 = jnp.sum(codebooks * codebooks, axis=2)
    grid = (n // _BLOCK,)
    indices, quantized = pl.pallas_call(
        _rvq_body,
        grid=grid,
        in_specs=[
            pl.BlockSpec((_BLOCK, d), lambda i: (i, 0)),
            pl.BlockSpec((n_cb, d, k), lambda i: (0, 0, 0)),
            pl.BlockSpec((n_cb, 3 * k, d), lambda i: (0, 0, 0)),
            pl.BlockSpec((n_cb, k), lambda i: (0, 0)),
        ],
        out_specs=[
            pl.BlockSpec((_BLOCK, n_cb), lambda i: (i, 0)),
            pl.BlockSpec((_BLOCK, d), lambda i: (i, 0)),
        ],
        out_shape=[
            jax.ShapeDtypeStruct((n, n_cb), jnp.int32),
            jax.ShapeDtypeStruct((n, d), jnp.float32),
        ],
    )(x, cbt, cbcat, cb2)
    return indices, quantized


# speedup vs baseline: 1.2170x; 1.2170x over previous
import jax
import jax.numpy as jnp
from jax import lax
from jax.experimental import pallas as pl

_BLOCK = 1024


def _rvq_body(x_ref, cbt_ref, cbcat_ref, cb2_ref, idx_ref, q_ref):
    resid = x_ref[...]
    quant = jnp.zeros_like(resid)
    n_cb = cbt_ref.shape[0]
    b, k = resid.shape[0], cbt_ref.shape[2]
    cols = []
    for i in range(n_cb):
        a2 = jnp.sum(resid * resid, axis=1, keepdims=True)
        b2 = cb2_ref[i, :][None, :]
        t = lax.dot_general(resid, cbt_ref[i],
                            (((1,), (0,)), ((), ())),
                            preferred_element_type=jnp.float32)
        d2 = a2 + b2 + t
        m2 = jnp.min(d2, axis=1, keepdims=True)
        s = jnp.sqrt(jnp.maximum(m2, 0.0))
        u0b = lax.bitcast_convert_type(s * s, jnp.int32)
        u = lax.bitcast_convert_type(u0b - 2, jnp.float32)
        for j in (-1, 0, 1, 2):
            uj = lax.bitcast_convert_type(u0b + j, jnp.float32)
            u = jnp.where(jnp.sqrt(uj) == s, uj, u)
        iota_k = lax.broadcasted_iota(jnp.int32, (b, k), 1)
        idx = jnp.min(jnp.where(d2 <= u, iota_k, k), axis=1)
        iota3 = lax.broadcasted_iota(jnp.int32, (b, 3 * k), 1)
        oh3 = ((iota3 & (k - 1)) == idx[:, None]).astype(jnp.bfloat16)
        q = lax.dot_general(oh3, cbcat_ref[i], (((1,), (0,)), ((), ())),
                            preferred_element_type=jnp.float32)
        quant = quant + q
        resid = resid - q
        cols.append(idx)
    idx_ref[...] = jnp.stack(cols, axis=1)
    q_ref[...] = quant


def kernel(x, codebooks):
    n, d = x.shape
    n_cb, k, _ = codebooks.shape
    cbt = -2.0 * jnp.swapaxes(codebooks, 1, 2)
    mask = jnp.uint32(0xFFFF0000)
    hi = lax.bitcast_convert_type(
        lax.bitcast_convert_type(codebooks, jnp.uint32) & mask, jnp.float32)
    r1 = codebooks - hi
    lo = lax.bitcast_convert_type(
        lax.bitcast_convert_type(r1, jnp.uint32) & mask, jnp.float32)
    lo2 = r1 - lo
    cbcat = jnp.concatenate(
        [hi.astype(jnp.bfloat16), lo.astype(jnp.bfloat16),
         lo2.astype(jnp.bfloat16)], axis=1)
    cb2 = jnp.sum(codebooks * codebooks, axis=2)
    grid = (n // _BLOCK,)
    indices, quantized = pl.pallas_call(
        _rvq_body,
        grid=grid,
        in_specs=[
            pl.BlockSpec((_BLOCK, d), lambda i: (i, 0)),
            pl.BlockSpec((n_cb, d, k), lambda i: (0, 0, 0)),
            pl.BlockSpec((n_cb, 3 * k, d), lambda i: (0, 0, 0)),
            pl.BlockSpec((n_cb, k), lambda i: (0, 0)),
        ],
        out_specs=[
            pl.BlockSpec((_BLOCK, n_cb), lambda i: (i, 0)),
            pl.BlockSpec((_BLOCK, d), lambda i: (i, 0)),
        ],
        out_shape=[
            jax.ShapeDtypeStruct((n, n_cb), jnp.int32),
            jax.ShapeDtypeStruct((n, d), jnp.float32),
        ],
    )(x, cbt, cbcat, cb2)
    return indices, quantized


# speedup vs baseline: 1.6044x; 1.3184x over previous
import jax
import jax.numpy as jnp
from jax import lax
from jax.experimental import pallas as pl

_BLOCK = 1024


_HALVES = 2


def _rvq_body(x_ref, cbt_ref, cbcat_ref, cb2_ref, idx_ref, q_ref):
    n_cb = cbt_ref.shape[0]
    k = cbt_ref.shape[2]
    bh = x_ref.shape[0] // _HALVES
    resids = [x_ref[h * bh:(h + 1) * bh, :] for h in range(_HALVES)]
    quants = [jnp.zeros_like(r) for r in resids]
    cols = [[] for _ in range(_HALVES)]
    iota_k = lax.broadcasted_iota(jnp.int32, (bh, k), 1)
    iota3 = lax.broadcasted_iota(jnp.int32, (bh, 3 * k), 1)
    for i in range(n_cb):
        b2 = cb2_ref[i, :][None, :]
        for h in range(_HALVES):
            resid = resids[h]
            a2 = jnp.sum(resid * resid, axis=1, keepdims=True)
            t = lax.dot_general(resid, cbt_ref[i],
                                (((1,), (0,)), ((), ())),
                                preferred_element_type=jnp.float32)
            d2 = a2 + b2 + t
            dist = jnp.sqrt(jnp.maximum(d2, 0.0))
            m = jnp.min(dist, axis=1, keepdims=True)
            idx = jnp.min(jnp.where(dist == m, iota_k, k), axis=1)
            oh3 = ((iota3 & (k - 1)) == idx[:, None]).astype(jnp.bfloat16)
            q = lax.dot_general(oh3, cbcat_ref[i], (((1,), (0,)), ((), ())),
                                preferred_element_type=jnp.float32)
            quants[h] = quants[h] + q
            resids[h] = resid - q
            cols[h].append(idx)
    for h in range(_HALVES):
        sl = slice(h * bh, (h + 1) * bh)
        idx_ref[sl, :] = jnp.stack(cols[h], axis=1)
        q_ref[sl, :] = quants[h]


def kernel(x, codebooks):
    n, d = x.shape
    n_cb, k, _ = codebooks.shape
    cbt = -2.0 * jnp.swapaxes(codebooks, 1, 2)
    mask = jnp.uint32(0xFFFF0000)
    hi = lax.bitcast_convert_type(
        lax.bitcast_convert_type(codebooks, jnp.uint32) & mask, jnp.float32)
    r1 = codebooks - hi
    lo = lax.bitcast_convert_type(
        lax.bitcast_convert_type(r1, jnp.uint32) & mask, jnp.float32)
    lo2 = r1 - lo
    cbcat = jnp.concatenate(
        [hi.astype(jnp.bfloat16), lo.astype(jnp.bfloat16),
         lo2.astype(jnp.bfloat16)], axis=1)
    cb2 = jnp.sum(codebooks * codebooks, axis=2)
    grid = (n // _BLOCK,)
    indices, quantized = pl.pallas_call(
        _rvq_body,
        grid=grid,
        in_specs=[
            pl.BlockSpec((_BLOCK, d), lambda i: (i, 0)),
            pl.BlockSpec((n_cb, d, k), lambda i: (0, 0, 0)),
            pl.BlockSpec((n_cb, 3 * k, d), lambda i: (0, 0, 0)),
            pl.BlockSpec((n_cb, k), lambda i: (0, 0)),
        ],
        out_specs=[
            pl.BlockSpec((_BLOCK, n_cb), lambda i: (i, 0)),
            pl.BlockSpec((_BLOCK, d), lambda i: (i, 0)),
        ],
        out_shape=[
            jax.ShapeDtypeStruct((n, n_cb), jnp.int32),
            jax.ShapeDtypeStruct((n, d), jnp.float32),
        ],
    )(x, cbt, cbcat, cb2)
    return indices, quantized
